# baseline (speedup 1.0000x reference)
import functools

import jax
import jax.numpy as jnp
from jax import lax
from jax.experimental import pallas as pl
from jax.experimental.pallas import tpu as pltpu
from jax.experimental.pallas import tpu_sc as plsc

_EPS = 1e-06
_TB = 512
_NB = 16


def _sc_mesh():
    return plsc.VectorSubcoreMesh(core_axis_name="c", subcore_axis_name="s")


def _worker_id():
    info = plsc.get_sparse_core_info()
    return lax.axis_index("s") * info.num_cores + lax.axis_index("c")


def _sc_gather(table, ids):
    info = plsc.get_sparse_core_info()
    nw = info.num_cores * info.num_subcores
    t, d = ids.shape[0], table.shape[1]
    b_per_w = t // nw

    @functools.partial(
        pl.kernel,
        mesh=_sc_mesh(),
        out_type=jax.ShapeDtypeStruct((t, d), table.dtype),
        scratch_types=[
            pltpu.VMEM((b_per_w,), jnp.int32),
            pltpu.VMEM((b_per_w, d), table.dtype),
            pltpu.SemaphoreType.DMA,
        ],
    )
    def gather_k(table_hbm, idx_hbm, out_hbm, idx_v, rows_v, sem):
        base = _worker_id() * b_per_w
        pltpu.sync_copy(idx_hbm.at[pl.ds(base, b_per_w)], idx_v)
        pltpu.async_copy(table_hbm.at[idx_v], rows_v, sem).wait()
        pltpu.sync_copy(rows_v, out_hbm.at[pl.ds(base, b_per_w)])

    return gather_k(table, ids)


def _route_body(h_ref, g_ref, wr_ref,
                xn_ref, p1_ref, p2_ref, w1_ref, w2_ref, be_ref, vc_ref):
    t, d = h_ref.shape
    e_num = wr_ref.shape[1]
    h = h_ref[...]
    ms = jnp.mean(h * h, axis=-1, keepdims=True)
    xn = h * lax.rsqrt(ms + _EPS) * (1.0 + g_ref[...])
    logits = jnp.dot(xn, wr_ref[...], preferred_element_type=jnp.float32)
    eidx = lax.broadcasted_iota(jnp.int32, logits.shape, 1)
    i1 = jnp.argmax(logits, axis=-1)[:, None].astype(jnp.int32)
    m1 = jnp.max(logits, axis=-1, keepdims=True)
    masked = jnp.where(eidx == i1, -jnp.inf, logits)
    i2 = jnp.argmax(masked, axis=-1)[:, None].astype(jnp.int32)
    m2 = jnp.max(masked, axis=-1, keepdims=True)
    bb = jnp.exp(m2 - m1)
    w1 = 1.0 / (1.0 + bb)
    w1_ref[...] = jnp.broadcast_to(w1, (t, 16))
    w2_ref[...] = jnp.broadcast_to(1.0 - w1, (t, 16))
    xn_ref[...] = xn.astype(jnp.bfloat16)

    a1 = (eidx == i1).astype(jnp.float32)
    a2 = (eidx == i2).astype(jnp.float32)
    a = a1 + a2
    rio = lax.broadcasted_iota(jnp.int32, (t, t), 0)
    cio = lax.broadcasted_iota(jnp.int32, (t, t), 1)
    tri = (cio < rio).astype(jnp.bfloat16)
    cex = jnp.dot(tri, a.astype(jnp.bfloat16),
                  preferred_element_type=jnp.float32)
    cnt = jnp.sum(a, axis=0, keepdims=True).astype(jnp.int32)
    nb = (cnt + _TB - 1) // _TB
    bs_parts, run = [], jnp.zeros((1, 1), jnp.int32)
    for e in range(e_num):
        bs_parts.append(run)
        run = run + nb[:, e:e + 1]
    bs = jnp.concatenate(bs_parts, axis=1)
    off = (bs * _TB).astype(jnp.float32)
    p1_ref[...] = jnp.sum(a1 * (off + cex), axis=1, keepdims=True).astype(jnp.int32)
    p2_ref[...] = jnp.sum(a2 * (off + cex), axis=1, keepdims=True).astype(jnp.int32)

    bs_col = jnp.concatenate([bs[:, e:e + 1] for e in range(e_num)], axis=0)
    nb_col = jnp.concatenate([nb[:, e:e + 1] for e in range(e_num)], axis=0)
    ct_col = jnp.concatenate([cnt[:, e:e + 1] for e in range(e_num)], axis=0)
    bio = lax.broadcasted_iota(jnp.int32, (e_num, _NB), 1)
    inblk = bio - bs_col
    owns = (inblk >= 0) & (inblk < nb_col)
    eio = lax.broadcasted_iota(jnp.int32, (e_num, _NB), 0)
    be = jnp.sum(jnp.where(owns, eio, 0), axis=0, keepdims=True)
    vc = jnp.sum(jnp.where(owns, jnp.clip(ct_col - inblk * _TB, 0, _TB), 0),
                 axis=0, keepdims=True)
    be_ref[...] = jnp.broadcast_to(be, (8, _NB))
    vc_ref[...] = jnp.broadcast_to(vc, (8, _NB))


def _route(h, gamma, w_router, *, interpret=False):
    t, d = h.shape
    e_num = w_router.shape[1]
    full = lambda *s: pl.BlockSpec(s, lambda: tuple(0 for _ in s))
    return pl.pallas_call(
        _route_body,
        in_specs=[full(t, d), full(1, d), full(d, e_num)],
        out_specs=[full(t, d), full(t, 1), full(t, 1), full(t, 16), full(t, 16),
                   full(8, _NB), full(8, _NB)],
        out_shape=[
            jax.ShapeDtypeStruct((t, d), jnp.bfloat16),
            jax.ShapeDtypeStruct((t, 1), jnp.int32),
            jax.ShapeDtypeStruct((t, 1), jnp.int32),
            jax.ShapeDtypeStruct((t, 16), jnp.float32),
            jax.ShapeDtypeStruct((t, 16), jnp.float32),
            jax.ShapeDtypeStruct((8, _NB), jnp.int32),
            jax.ShapeDtypeStruct((8, _NB), jnp.int32),
        ],
        interpret=interpret,
    )(h, gamma, w_router)


def _sc_dispatch(xn_i, pos1, pos2):
    info = plsc.get_sparse_core_info()
    nw = info.num_cores * info.num_subcores
    t, d2 = xn_i.shape
    b_per_w = t // nw

    @functools.partial(
        pl.kernel,
        mesh=_sc_mesh(),
        out_type=jax.ShapeDtypeStruct((_NB * _TB, d2), jnp.int32),
        scratch_types=[
            pltpu.VMEM((b_per_w,), jnp.int32),
            pltpu.VMEM((b_per_w,), jnp.int32),
            pltpu.VMEM((b_per_w, d2), jnp.int32),
            pltpu.SemaphoreType.DMA,
        ],
    )
    def dispatch_k(xn_hbm, p1_hbm, p2_hbm, xd_hbm, p1_v, p2_v, rows_v, sem):
        base = _worker_id() * b_per_w
        pltpu.sync_copy(xn_hbm.at[pl.ds(base, b_per_w)], rows_v)
        pltpu.sync_copy(p1_hbm.at[pl.ds(base, b_per_w)], p1_v)
        pltpu.sync_copy(p2_hbm.at[pl.ds(base, b_per_w)], p2_v)
        pltpu.async_copy(rows_v, xd_hbm.at[p1_v], sem).wait()
        pltpu.async_copy(rows_v, xd_hbm.at[p2_v], sem).wait()

    return dispatch_k(xn_i, pos1, pos2)


def _experts_body(be_ref, vc_ref, xd_ref, wg_ref, wu_ref, wd_ref, out_ref):
    b = pl.program_id(0)

    @pl.when(vc_ref[b] > 0)
    def _():
        x = xd_ref[...]
        g = jnp.dot(x, wg_ref[0], preferred_element_type=jnp.float32)
        u = jnp.dot(x, wu_ref[0], preferred_element_type=jnp.float32)
        y = (g * jax.nn.sigmoid(g) * u).astype(jnp.bfloat16)
        out_ref[...] = jnp.dot(y, wd_ref[0], preferred_element_type=jnp.float32)


def _experts(xd, be, vc, wg, wu, wd, *, interpret=False):
    n, d = xd.shape
    e_num, _, f = wg.shape
    grid_spec = pltpu.PrefetchScalarGridSpec(
        num_scalar_prefetch=2,
        grid=(_NB,),
        in_specs=[
            pl.BlockSpec((_TB, d), lambda b, be_r, vc_r: (b, 0)),
            pl.BlockSpec((1, d, f), lambda b, be_r, vc_r: (be_r[b], 0, 0)),
            pl.BlockSpec((1, d, f), lambda b, be_r, vc_r: (be_r[b], 0, 0)),
            pl.BlockSpec((1, f, d), lambda b, be_r, vc_r: (be_r[b], 0, 0)),
        ],
        out_specs=pl.BlockSpec((_TB, d), lambda b, be_r, vc_r: (b, 0)),
    )
    return pl.pallas_call(
        _experts_body,
        grid_spec=grid_spec,
        out_shape=jax.ShapeDtypeStruct((n, d), jnp.float32),
        interpret=interpret,
    )(be, vc, xd, wg, wu, wd)


def _sc_combine(h, yp, pos1, pos2, w1, w2):
    info = plsc.get_sparse_core_info()
    nw = info.num_cores * info.num_subcores
    t, d = h.shape
    b_per_w = t // nw
    ck = 16
    n_ck = b_per_w // ck

    @functools.partial(
        pl.kernel,
        mesh=_sc_mesh(),
        out_type=jax.ShapeDtypeStruct((t, d), jnp.float32),
        scratch_types=[
            pltpu.VMEM((ck,), jnp.int32),
            pltpu.VMEM((ck,), jnp.int32),
            pltpu.VMEM((ck, 16), jnp.float32),
            pltpu.VMEM((ck, 16), jnp.float32),
            pltpu.VMEM((ck, d), jnp.float32),
            pltpu.VMEM((ck, d), jnp.float32),
            pltpu.VMEM((ck, d), jnp.float32),
            pltpu.VMEM((ck, d), jnp.float32),
            pltpu.SemaphoreType.DMA,
        ],
    )
    def combine_k(h_hbm, yp_hbm, p1_hbm, p2_hbm, w1_hbm, w2_hbm, out_hbm,
                  i1_v, i2_v, s1_v, s2_v, y1_v, y2_v, h_v, o_v, sem):
        base = _worker_id() * b_per_w
        for c in range(n_ck):
            cb = base + c * ck
            pltpu.sync_copy(p1_hbm.at[pl.ds(cb, ck)], i1_v)
            pltpu.sync_copy(p2_hbm.at[pl.ds(cb, ck)], i2_v)
            pltpu.sync_copy(w1_hbm.at[pl.ds(cb, ck)], s1_v)
            pltpu.sync_copy(w2_hbm.at[pl.ds(cb, ck)], s2_v)
            pltpu.sync_copy(h_hbm.at[pl.ds(cb, ck)], h_v)
            pltpu.async_copy(yp_hbm.at[i1_v], y1_v, sem).wait()
            pltpu.async_copy(yp_hbm.at[i2_v], y2_v, sem).wait()
            for i in range(ck):
                s1 = s1_v[i]
                s2 = s2_v[i]

                def body(jv, _, i=i, s1=s1, s2=s2):
                    sl = pl.ds(jv * 16, 16)
                    o_v[i, sl] = h_v[i, sl] + s1 * y1_v[i, sl] + s2 * y2_v[i, sl]
                    return 0

                lax.fori_loop(0, d // 16, body, 0)
            pltpu.sync_copy(o_v, out_hbm.at[pl.ds(cb, ck)])

    return combine_k(h, yp, pos1, pos2, w1, w2)


def kernel(input_ids, embed_table, norm_gamma, w_router, w_gate, w_up, w_down):
    b, s = input_ids.shape
    t = b * s
    d = embed_table.shape[1]
    ids = input_ids.reshape(-1).astype(jnp.int32)
    h = _sc_gather(embed_table, ids)
    xn, p1, p2, w1, w2, be, vc = _route(h, norm_gamma.reshape(1, d), w_router)
    xn_i = lax.bitcast_convert_type(xn.reshape(t, d // 2, 2), jnp.int32)
    xd_i = _sc_dispatch(xn_i, p1.reshape(-1), p2.reshape(-1))
    xd = lax.bitcast_convert_type(xd_i, jnp.bfloat16).reshape(_NB * _TB, d)
    yp = _experts(xd, be[0], vc[0],
                  w_gate.astype(jnp.bfloat16), w_up.astype(jnp.bfloat16),
                  w_down.astype(jnp.bfloat16))
    out = _sc_combine(h, yp, p1.reshape(-1), p2.reshape(-1), w1, w2)
    return out.reshape(b, s, d)


# speedup vs baseline: 1.8527x; 1.8527x over previous
import functools

import jax
import jax.numpy as jnp
from jax import lax
from jax.experimental import pallas as pl
from jax.experimental.pallas import tpu as pltpu
from jax.experimental.pallas import tpu_sc as plsc

_EPS = 1e-06


def _sc_mesh():
    return plsc.VectorSubcoreMesh(core_axis_name="c", subcore_axis_name="s")


def _worker_id():
    info = plsc.get_sparse_core_info()
    return lax.axis_index("s") * info.num_cores + lax.axis_index("c")


def _sc_gather(table, ids):
    info = plsc.get_sparse_core_info()
    nw = info.num_cores * info.num_subcores
    t, d = ids.shape[0], table.shape[1]
    b_per_w = t // nw

    @functools.partial(
        pl.kernel,
        mesh=_sc_mesh(),
        out_type=jax.ShapeDtypeStruct((t, d), table.dtype),
        scratch_types=[
            pltpu.VMEM((b_per_w,), jnp.int32),
            pltpu.VMEM((b_per_w, d), table.dtype),
            pltpu.SemaphoreType.DMA,
        ],
    )
    def gather_k(table_hbm, idx_hbm, out_hbm, idx_v, rows_v, sem):
        base = _worker_id() * b_per_w
        pltpu.sync_copy(idx_hbm.at[pl.ds(base, b_per_w)], idx_v)
        pltpu.async_copy(table_hbm.at[idx_v], rows_v, sem).wait()
        pltpu.sync_copy(rows_v, out_hbm.at[pl.ds(base, b_per_w)])

    return gather_k(table, ids)


def _route_body(h_ref, g_ref, wr_ref,
                xn_ref, i1_ref, i2_ref, w1_ref, w2_ref):
    h = h_ref[...]
    ms = jnp.mean(h * h, axis=-1, keepdims=True)
    xn = h * lax.rsqrt(ms + _EPS) * (1.0 + g_ref[...])
    logits = jnp.dot(xn, wr_ref[...], preferred_element_type=jnp.float32)
    eidx = lax.broadcasted_iota(jnp.int32, logits.shape, 1)
    i1 = jnp.argmax(logits, axis=-1)[:, None].astype(jnp.int32)
    m1 = jnp.max(logits, axis=-1, keepdims=True)
    masked = jnp.where(eidx == i1, -jnp.inf, logits)
    i2 = jnp.argmax(masked, axis=-1)[:, None].astype(jnp.int32)
    m2 = jnp.max(masked, axis=-1, keepdims=True)
    bb = jnp.exp(m2 - m1)
    w1 = 1.0 / (1.0 + bb)
    i1_ref[...] = i1
    i2_ref[...] = i2
    w1_ref[...] = w1
    w2_ref[...] = 1.0 - w1
    xn_ref[...] = xn.astype(jnp.bfloat16)


def _route(h, gamma, w_router, *, interpret=False):
    t, d = h.shape
    e_num = w_router.shape[1]
    full = lambda *s: pl.BlockSpec(s, lambda: tuple(0 for _ in s))
    return pl.pallas_call(
        _route_body,
        in_specs=[full(t, d), full(1, d), full(d, e_num)],
        out_specs=[full(t, d), full(t, 1), full(t, 1), full(t, 1), full(t, 1)],
        out_shape=[
            jax.ShapeDtypeStruct((t, d), jnp.bfloat16),
            jax.ShapeDtypeStruct((t, 1), jnp.int32),
            jax.ShapeDtypeStruct((t, 1), jnp.int32),
            jax.ShapeDtypeStruct((t, 1), jnp.float32),
            jax.ShapeDtypeStruct((t, 1), jnp.float32),
        ],
        interpret=interpret,
    )(h, gamma, w_router)


def _moe_body(xn_ref, h_ref, i1_ref, i2_ref, w1_ref, w2_ref,
              wg_ref, wu_ref, wd_ref, out_ref, acc_ref, *, tb, e_num):
    e = pl.program_id(0)
    i = pl.program_id(1)
    sl = pl.ds(i * tb, tb)
    x = xn_ref[sl, :]
    g = jnp.dot(x, wg_ref[0], preferred_element_type=jnp.float32)
    u = jnp.dot(x, wu_ref[0], preferred_element_type=jnp.float32)
    ge = g * jax.nn.sigmoid(g) * u
    wsel = (w1_ref[sl, :] * (i1_ref[sl, :] == e).astype(jnp.float32)
            + w2_ref[sl, :] * (i2_ref[sl, :] == e).astype(jnp.float32))
    yw = (ge * wsel).astype(jnp.bfloat16)
    contrib = jnp.dot(yw, wd_ref[0], preferred_element_type=jnp.float32)

    @pl.when(e == 0)
    def _init():
        acc_ref[sl, :] = h_ref[sl, :] + contrib

    @pl.when(e > 0)
    def _acc():
        acc_ref[sl, :] += contrib

    @pl.when(e == e_num - 1)
    def _emit():
        out_ref[sl, :] = acc_ref[sl, :]


def _moe(xn, h, i1, i2, w1, w2, wg, wu, wd, *, interpret=False, nt=2):
    t, d = h.shape
    e_num, _, f = wg.shape
    tb = t // nt
    full = lambda *s: pl.BlockSpec(s, lambda e, i: tuple(0 for _ in s))
    return pl.pallas_call(
        functools.partial(_moe_body, tb=tb, e_num=e_num),
        grid=(e_num, nt),
        in_specs=[
            full(t, d),
            full(t, d),
            full(t, 1),
            full(t, 1),
            full(t, 1),
            full(t, 1),
            pl.BlockSpec((1, d, f), lambda e, i: (e, 0, 0)),
            pl.BlockSpec((1, d, f), lambda e, i: (e, 0, 0)),
            pl.BlockSpec((1, f, d), lambda e, i: (e, 0, 0)),
        ],
        out_specs=full(t, d),
        out_shape=jax.ShapeDtypeStruct((t, d), jnp.float32),
        scratch_shapes=[pltpu.VMEM((t, d), jnp.float32)],
        interpret=interpret,
    )(xn, h, i1, i2, w1, w2, wg, wu, wd)


def kernel(input_ids, embed_table, norm_gamma, w_router, w_gate, w_up, w_down):
    b, s = input_ids.shape
    d = embed_table.shape[1]
    ids = input_ids.reshape(-1).astype(jnp.int32)
    h = _sc_gather(embed_table, ids)
    xn, i1, i2, w1, w2 = _route(h, norm_gamma.reshape(1, d), w_router)
    out = _moe(xn, h, i1, i2, w1, w2,
               w_gate.astype(jnp.bfloat16), w_up.astype(jnp.bfloat16),
               w_down.astype(jnp.bfloat16))
    return out.reshape(b, s, d)
